# Initial kernel scaffold; baseline (speedup 1.0000x reference)
import jax
import jax.numpy as jnp
from jax.experimental import pallas as pl


def kernel(x, edge_index, batch, W1, att_src, att_dst, b1, lin_W, lin_b):
    raise NotImplementedError("write your pallas kernel here")


# speedup vs baseline: 50.5322x; 50.5322x over previous
import dataclasses
import functools

import jax
import jax.numpy as jnp
from jax import lax
from jax.experimental import pallas as pl
from jax.experimental.pallas import tpu as pltpu
from jax.experimental.pallas import tpu_sc as plsc

N = 10240
E = 327680
IN_CH = 128
HID = 64
NPG = 128
G = N // NPG
W80 = 128
NC = 2
NS = 16
NW = NC * NS
EPW = E // NW
CK = 128
NCHUNK = EPW // CK
SB = 8
NBLK = NCHUNK // SB
RPT = N // NS


def _tc_prep(x_ref, w_ref, asrc_ref, adst_ref, h80_ref, self80_ref, alpha_ref):
    h = jnp.dot(x_ref[...], w_ref[...], preferred_element_type=jnp.float32)
    a_s = jnp.sum(h * asrc_ref[...], axis=1)
    a_d = jnp.sum(h * adst_ref[...], axis=1)
    e = a_s + a_d
    w_self = jnp.exp(jnp.where(e >= 0.0, e, 0.2 * e))
    h80 = jnp.concatenate(
        [h, jnp.ones((h.shape[0], 1), jnp.float32),
         jnp.zeros((h.shape[0], W80 - HID - 1), jnp.float32)], axis=1)
    h80_ref[...] = h80
    self80_ref[...] = h80 * w_self[:, None]
    alpha_ref[...] = jnp.stack([a_s, a_d], axis=0)


def _sc_edges(h80_hbm, alpha2_hbm, src_hbm, dst_hbm, out_hbm,
              src_v, dst_v, asrc_v, adst_v, rows_v, w_v, acc_sh, sem):
    cid = lax.axis_index("c")
    sid = lax.axis_index("s")
    wid = cid * NS + sid

    pltpu.sync_copy(alpha2_hbm.at[0], asrc_v)
    pltpu.sync_copy(alpha2_hbm.at[1], adst_v)

    @pl.loop(0, CK)
    def _zero_rows(j):
        for c in range(W80 // 16):
            rows_v[j, pl.ds(c * 16, 16)] = jnp.zeros((16,), jnp.float32)

    @pl.loop(0, RPT // CK)
    def _zero_acc(k):
        pltpu.sync_copy(rows_v, acc_sh.at[pl.ds(sid * RPT + k * CK, CK)])

    plsc.subcore_barrier()

    @pl.loop(0, NBLK)
    def _block(blk):
        pltpu.sync_copy(src_hbm.at[wid, pl.ds(blk * SB, SB)], src_v)
        pltpu.sync_copy(dst_hbm.at[wid, pl.ds(blk * SB, SB)], dst_v)

        for b in range(SB):
            pltpu.async_copy(h80_hbm.at[src_v.at[b]], rows_v, sem).wait()

            for i in range(CK // 16):
                sv = src_v[b, pl.ds(i * 16, 16)]
                dv = dst_v[b, pl.ds(i * 16, 16)]
                e = (plsc.load_gather(asrc_v, [sv])
                     + plsc.load_gather(adst_v, [dv]))
                e = jnp.where(e >= 0.0, e, 0.2 * e)
                w_v[pl.ds(i * 16, 16)] = jnp.exp(e)

            @pl.loop(0, CK // 16)
            def _scale(i):
                wv = w_v[pl.ds(i * 16, 16)]
                for j in range(16):
                    ws = wv[j]
                    r = i * 16 + j
                    for c in range(W80 // 16):
                        rows_v[r, pl.ds(c * 16, 16)] = (
                            rows_v[r, pl.ds(c * 16, 16)] * ws)

            pltpu.sync_copy(rows_v, acc_sh.at[dst_v.at[b]], add=True)

    plsc.subcore_barrier()

    pltpu.sync_copy(acc_sh.at[pl.ds(sid * RPT, RPT)],
                    out_hbm.at[cid, pl.ds(sid * RPT, RPT)])


GB = 8


def _tc_head(parts_ref, self_ref, b1_ref, w2_ref, lb_ref, out_ref):
    tot = parts_ref[0] + parts_ref[1] + self_ref[...]
    act = jnp.maximum(tot[:, :HID] / tot[:, HID:HID + 1] + b1_ref[...], 0.0)
    prod = act * w2_ref[...]
    y = jnp.sum(prod.reshape(GB, NPG, HID), axis=(1, 2)) + lb_ref[0, 0]
    out_ref[...] = (1.0 / (1.0 + jnp.exp(-y)))[:, None]


def kernel(x, edge_index, batch, W1, att_src, att_dst, b1, lin_W, lin_b):
    src3 = edge_index[0].reshape(NW, NCHUNK, CK).astype(jnp.int32)
    dst3 = edge_index[1].reshape(NW, NCHUNK, CK).astype(jnp.int32)

    h80, self80, alpha2 = pl.pallas_call(
        _tc_prep,
        out_shape=[
            jax.ShapeDtypeStruct((N, W80), jnp.float32),
            jax.ShapeDtypeStruct((N, W80), jnp.float32),
            jax.ShapeDtypeStruct((2, N), jnp.float32),
        ],
    )(x, W1, att_src.reshape(1, HID), att_dst.reshape(1, HID))

    cp = pltpu.CompilerParams()
    if "needs_layout_passes" in pltpu.CompilerParams.__dataclass_fields__:
        cp = dataclasses.replace(cp, needs_layout_passes=False)
    sc_fn = functools.partial(
        pl.kernel,
        compiler_params=cp,
        out_type=jax.ShapeDtypeStruct((NC, N, W80), jnp.float32),
        mesh=plsc.VectorSubcoreMesh(core_axis_name="c", subcore_axis_name="s"),
        scratch_types=[
            pltpu.VMEM((SB, CK), jnp.int32),
            pltpu.VMEM((SB, CK), jnp.int32),
            pltpu.VMEM((N,), jnp.float32),
            pltpu.VMEM((N,), jnp.float32),
            pltpu.VMEM((CK, W80), jnp.float32),
            pltpu.VMEM((CK,), jnp.float32),
            pltpu.VMEM_SHARED((N, W80), jnp.float32),
            pltpu.SemaphoreType.DMA,
        ],
    )(_sc_edges)
    parts = sc_fn(h80, alpha2, src3, dst3)

    w2t = jnp.tile(lin_W.reshape(NPG, HID), (GB, 1))
    out = pl.pallas_call(
        _tc_head,
        grid=(G // GB,),
        in_specs=[
            pl.BlockSpec((NC, GB * NPG, W80), lambda g: (0, g, 0)),
            pl.BlockSpec((GB * NPG, W80), lambda g: (g, 0)),
            pl.BlockSpec((1, HID), lambda g: (0, 0)),
            pl.BlockSpec((GB * NPG, HID), lambda g: (0, 0)),
            pl.BlockSpec((1, 1), lambda g: (0, 0)),
        ],
        out_specs=pl.BlockSpec((GB, 1), lambda g: (g, 0)),
        out_shape=jax.ShapeDtypeStruct((G, 1), jnp.float32),
    )(parts, self80, b1.reshape(1, HID), w2t, lin_b.reshape(1, 1))
    return out


# speedup vs baseline: 79.5229x; 1.5737x over previous
import dataclasses
import functools

import jax
import jax.numpy as jnp
from jax import lax
from jax.experimental import pallas as pl
from jax.experimental.pallas import tpu as pltpu
from jax.experimental.pallas import tpu_sc as plsc

N = 10240
E = 327680
IN_CH = 128
HID = 64
NPG = 128
G = N // NPG
WROW = 128
WSELF = 80
NC = 2
NS = 16
NW = NC * NS
EPW = E // NW
CK = 128
NCHUNK = EPW // CK
MB = 16
RPT = N // NS


def _tc_prep(x_ref, w_ref, asrc_ref, adst_ref, h128_ref, self_ref,
             alpha_ref):
    h = jnp.dot(x_ref[...], w_ref[...], preferred_element_type=jnp.float32)
    a_s = jnp.sum(h * asrc_ref[...], axis=1)
    a_d = jnp.sum(h * adst_ref[...], axis=1)
    e = a_s + a_d
    w_self = jnp.exp(jnp.where(e >= 0.0, e, 0.2 * e))
    n = h.shape[0]
    ones = jnp.ones((n, 1), jnp.float32)
    h128_ref[...] = jnp.concatenate(
        [h, ones, a_s[:, None],
         jnp.zeros((n, WROW - HID - 2), jnp.float32)], axis=1)
    h80 = jnp.concatenate(
        [h, ones, jnp.zeros((n, WSELF - HID - 1), jnp.float32)], axis=1)
    self_ref[...] = h80 * w_self[:, None]
    alpha_ref[...] = jnp.stack([a_s, a_d], axis=0)


def _sc_edges(h128_hbm, alpha2_hbm, src_hbm, dst_hbm, out_hbm,
              src_v, dst_v, adst_v, rows_a, rows_b, acc_sh, sem_ga, sem_gb):
    cid = lax.axis_index("c")
    sid = lax.axis_index("s")
    wid = cid * NS + sid

    pltpu.sync_copy(alpha2_hbm.at[1], adst_v)

    @pl.loop(0, CK)
    def _zero_rows(j):
        for c in range(WROW // 16):
            rows_a[j, pl.ds(c * 16, 16)] = jnp.zeros((16,), jnp.float32)

    @pl.loop(0, RPT // CK)
    def _zero_acc(k):
        pltpu.sync_copy(rows_a, acc_sh.at[pl.ds(sid * RPT + k * CK, CK)])

    plsc.subcore_barrier()

    def gather_start(lg, buf, sem):
        pltpu.async_copy(h128_hbm.at[src_v.at[lg]], buf, sem)

    def gather_wait(lg, buf, sem):
        pltpu.make_async_copy(h128_hbm.at[src_v.at[lg]], buf, sem).wait()

    def process(lg, buf, gsem, prefetch):
        gather_wait(lg, buf, gsem)

        @pl.loop(0, CK // 16)
        def _group(i):
            dv = dst_v[lg, pl.ds(i * 16, 16)]
            rvec = lax.iota(jnp.int32, 16) + i * 16
            cvec = jnp.full((16,), HID + 1, jnp.int32)
            e = (plsc.load_gather(buf, [rvec, cvec])
                 + plsc.load_gather(adst_v, [dv]))
            e = jnp.where(e >= 0.0, e, 0.2 * e)
            wv = jnp.exp(e)
            for j in range(16):
                ws = wv[j]
                r = i * 16 + j
                for c in range(5):
                    buf[r, pl.ds(c * 16, 16)] = (
                        buf[r, pl.ds(c * 16, 16)] * ws)

        pltpu.sync_copy(buf, acc_sh.at[dst_v.at[lg]], add=True)

        if prefetch:
            gather_start(lg + 2, buf, gsem)

    @pl.loop(0, NCHUNK // MB)
    def _block(mb):
        pltpu.sync_copy(src_hbm.at[wid, pl.ds(mb * MB, MB)], src_v)
        pltpu.sync_copy(dst_hbm.at[wid, pl.ds(mb * MB, MB)], dst_v)
        gather_start(0, rows_a, sem_ga)
        gather_start(1, rows_b, sem_gb)

        @pl.loop(0, MB // 2 - 1)
        def _pair(p):
            process(p * 2, rows_a, sem_ga, True)
            process(p * 2 + 1, rows_b, sem_gb, True)

        process(MB - 2, rows_a, sem_ga, False)
        process(MB - 1, rows_b, sem_gb, False)

    plsc.subcore_barrier()

    pltpu.sync_copy(acc_sh.at[pl.ds(sid * RPT, RPT)],
                    out_hbm.at[cid, pl.ds(sid * RPT, RPT)])


GB = 8


def _tc_head(parts_ref, self_ref, b1_ref, w2_ref, lb_ref, out_ref):
    tot = parts_ref[0] + parts_ref[1]
    num = tot[:, :HID] + self_ref[:, :HID]
    den = tot[:, HID:HID + 1] + self_ref[:, HID:HID + 1]
    act = jnp.maximum(num / den + b1_ref[...], 0.0)
    prod = act * w2_ref[...]
    y = jnp.sum(prod.reshape(GB, NPG, HID), axis=(1, 2)) + lb_ref[0, 0]
    out_ref[...] = (1.0 / (1.0 + jnp.exp(-y)))[:, None]


def kernel(x, edge_index, batch, W1, att_src, att_dst, b1, lin_W, lin_b):
    src3 = edge_index[0].reshape(NW, NCHUNK, CK).astype(jnp.int32)
    dst3 = edge_index[1].reshape(NW, NCHUNK, CK).astype(jnp.int32)

    h128, self80, alpha2 = pl.pallas_call(
        _tc_prep,
        out_shape=[
            jax.ShapeDtypeStruct((N, WROW), jnp.float32),
            jax.ShapeDtypeStruct((N, WSELF), jnp.float32),
            jax.ShapeDtypeStruct((2, N), jnp.float32),
        ],
    )(x, W1, att_src.reshape(1, HID), att_dst.reshape(1, HID))

    cp = pltpu.CompilerParams()
    if "needs_layout_passes" in pltpu.CompilerParams.__dataclass_fields__:
        cp = dataclasses.replace(cp, needs_layout_passes=False)
    sc_fn = functools.partial(
        pl.kernel,
        compiler_params=cp,
        out_type=jax.ShapeDtypeStruct((NC, N, WROW), jnp.float32),
        mesh=plsc.VectorSubcoreMesh(core_axis_name="c", subcore_axis_name="s"),
        scratch_types=[
            pltpu.VMEM((MB, CK), jnp.int32),
            pltpu.VMEM((MB, CK), jnp.int32),
            pltpu.VMEM((N,), jnp.float32),
            pltpu.VMEM((CK, WROW), jnp.float32),
            pltpu.VMEM((CK, WROW), jnp.float32),
            pltpu.VMEM_SHARED((N, WROW), jnp.float32),
            pltpu.SemaphoreType.DMA,
            pltpu.SemaphoreType.DMA,
        ],
    )(_sc_edges)
    parts = sc_fn(h128, alpha2, src3, dst3)

    w2t = jnp.tile(lin_W.reshape(NPG, HID), (GB, 1))
    out = pl.pallas_call(
        _tc_head,
        grid=(G // GB,),
        in_specs=[
            pl.BlockSpec((NC, GB * NPG, WROW), lambda g: (0, g, 0)),
            pl.BlockSpec((GB * NPG, WSELF), lambda g: (g, 0)),
            pl.BlockSpec((1, HID), lambda g: (0, 0)),
            pl.BlockSpec((GB * NPG, HID), lambda g: (0, 0)),
            pl.BlockSpec((1, 1), lambda g: (0, 0)),
        ],
        out_specs=pl.BlockSpec((GB, 1), lambda g: (g, 0)),
        out_shape=jax.ShapeDtypeStruct((G, 1), jnp.float32),
    )(parts, self80, b1.reshape(1, HID), w2t, lin_b.reshape(1, 1))
    return out


# speedup vs baseline: 80.2779x; 1.0095x over previous
import dataclasses
import functools

import jax
import jax.numpy as jnp
from jax import lax
from jax.experimental import pallas as pl
from jax.experimental.pallas import tpu as pltpu
from jax.experimental.pallas import tpu_sc as plsc

N = 10240
E = 327680
IN_CH = 128
HID = 64
NPG = 128
G = N // NPG
WROW = 128
WSELF = 80
NC = 2
NS = 16
NW = NC * NS
EPW = E // NW
CK = 128
NCHUNK = EPW // CK
MB = 16
RPT = N // NS


def _tc_prep(x_ref, w_ref, asrc_ref, adst_ref, h128_ref, self_ref,
             alpha_ref):
    h = jnp.dot(x_ref[...], w_ref[...], preferred_element_type=jnp.float32)
    a_s = jnp.sum(h * asrc_ref[...], axis=1)
    a_d = jnp.sum(h * adst_ref[...], axis=1)
    e = a_s + a_d
    w_self = jnp.exp(jnp.where(e >= 0.0, e, 0.2 * e))
    n = h.shape[0]
    ones = jnp.ones((n, 1), jnp.float32)
    h128_ref[...] = jnp.concatenate(
        [h, ones, a_s[:, None],
         jnp.zeros((n, WROW - HID - 2), jnp.float32)], axis=1)
    h80 = jnp.concatenate(
        [h, ones, jnp.zeros((n, WSELF - HID - 1), jnp.float32)], axis=1)
    self_ref[...] = h80 * w_self[:, None]
    alpha_ref[...] = jnp.stack([a_s, a_d], axis=0)


def _sc_edges(h128_hbm, alpha2_hbm, src_hbm, dst_hbm, out_hbm,
              src_v, dst_v, adst_v, rows_a, rows_b, acc_sh, sem_ga, sem_gb):
    cid = lax.axis_index("c")
    sid = lax.axis_index("s")
    wid = cid * NS + sid

    pltpu.sync_copy(alpha2_hbm.at[1], adst_v)

    @pl.loop(0, CK)
    def _zero_rows(j):
        for c in range(WROW // 16):
            rows_a[j, pl.ds(c * 16, 16)] = jnp.zeros((16,), jnp.float32)

    @pl.loop(0, RPT // CK)
    def _zero_acc(k):
        pltpu.sync_copy(rows_a, acc_sh.at[pl.ds(sid * RPT + k * CK, CK)])

    plsc.subcore_barrier()

    def gather_start(lg, buf, sem):
        pltpu.async_copy(h128_hbm.at[src_v.at[lg]], buf, sem)

    def gather_wait(lg, buf, sem):
        pltpu.make_async_copy(h128_hbm.at[src_v.at[lg]], buf, sem).wait()

    def process(lg, buf, gsem, prefetch):
        gather_wait(lg, buf, gsem)

        @pl.loop(0, CK // 16)
        def _group(i):
            dv = dst_v[lg, pl.ds(i * 16, 16)]
            rvec = lax.iota(jnp.int32, 16) + i * 16
            cvec = jnp.full((16,), HID + 1, jnp.int32)
            e = (plsc.load_gather(buf, [rvec, cvec])
                 + plsc.load_gather(adst_v, [dv]))
            e = jnp.where(e >= 0.0, e, 0.2 * e)
            wv = jnp.exp(e)
            for j in range(16):
                ws = wv[j]
                r = i * 16 + j
                for c in range(4):
                    buf[r, pl.ds(c * 16, 16)] = (
                        buf[r, pl.ds(c * 16, 16)] * ws)
                buf[r, pl.ds(HID, 16)] = jnp.full((16,), ws, jnp.float32)

        pltpu.sync_copy(buf, acc_sh.at[dst_v.at[lg]], add=True)

        if prefetch:
            gather_start(lg + 2, buf, gsem)

    @pl.loop(0, NCHUNK // MB)
    def _block(mb):
        pltpu.sync_copy(src_hbm.at[wid, pl.ds(mb * MB, MB)], src_v)
        pltpu.sync_copy(dst_hbm.at[wid, pl.ds(mb * MB, MB)], dst_v)
        gather_start(0, rows_a, sem_ga)
        gather_start(1, rows_b, sem_gb)

        @pl.loop(0, MB // 2 - 1)
        def _pair(p):
            process(p * 2, rows_a, sem_ga, True)
            process(p * 2 + 1, rows_b, sem_gb, True)

        process(MB - 2, rows_a, sem_ga, False)
        process(MB - 1, rows_b, sem_gb, False)

    plsc.subcore_barrier()

    pltpu.sync_copy(acc_sh.at[pl.ds(sid * RPT, RPT)],
                    out_hbm.at[cid, pl.ds(sid * RPT, RPT)])


GB = 8


def _tc_head(parts_ref, self_ref, b1_ref, w2_ref, lb_ref, out_ref):
    tot = parts_ref[0] + parts_ref[1]
    num = tot[:, :HID] + self_ref[:, :HID]
    den = tot[:, HID:HID + 1] + self_ref[:, HID:HID + 1]
    act = jnp.maximum(num / den + b1_ref[...], 0.0)
    prod = act * w2_ref[...]
    y = jnp.sum(prod.reshape(GB, NPG, HID), axis=(1, 2)) + lb_ref[0, 0]
    out_ref[...] = (1.0 / (1.0 + jnp.exp(-y)))[:, None]


def kernel(x, edge_index, batch, W1, att_src, att_dst, b1, lin_W, lin_b):
    src3 = edge_index[0].reshape(NW, NCHUNK, CK).astype(jnp.int32)
    dst3 = edge_index[1].reshape(NW, NCHUNK, CK).astype(jnp.int32)

    h128, self80, alpha2 = pl.pallas_call(
        _tc_prep,
        out_shape=[
            jax.ShapeDtypeStruct((N, WROW), jnp.float32),
            jax.ShapeDtypeStruct((N, WSELF), jnp.float32),
            jax.ShapeDtypeStruct((2, N), jnp.float32),
        ],
    )(x, W1, att_src.reshape(1, HID), att_dst.reshape(1, HID))

    cp = pltpu.CompilerParams()
    if "needs_layout_passes" in pltpu.CompilerParams.__dataclass_fields__:
        cp = dataclasses.replace(cp, needs_layout_passes=False)
    sc_fn = functools.partial(
        pl.kernel,
        compiler_params=cp,
        out_type=jax.ShapeDtypeStruct((NC, N, WROW), jnp.float32),
        mesh=plsc.VectorSubcoreMesh(core_axis_name="c", subcore_axis_name="s"),
        scratch_types=[
            pltpu.VMEM((MB, CK), jnp.int32),
            pltpu.VMEM((MB, CK), jnp.int32),
            pltpu.VMEM((N,), jnp.float32),
            pltpu.VMEM((CK, WROW), jnp.float32),
            pltpu.VMEM((CK, WROW), jnp.float32),
            pltpu.VMEM_SHARED((N, WROW), jnp.float32),
            pltpu.SemaphoreType.DMA,
            pltpu.SemaphoreType.DMA,
        ],
    )(_sc_edges)
    parts = sc_fn(h128, alpha2, src3, dst3)

    w2t = jnp.tile(lin_W.reshape(NPG, HID), (GB, 1))
    out = pl.pallas_call(
        _tc_head,
        grid=(G // GB,),
        in_specs=[
            pl.BlockSpec((NC, GB * NPG, WROW), lambda g: (0, g, 0)),
            pl.BlockSpec((GB * NPG, WSELF), lambda g: (g, 0)),
            pl.BlockSpec((1, HID), lambda g: (0, 0)),
            pl.BlockSpec((GB * NPG, HID), lambda g: (0, 0)),
            pl.BlockSpec((1, 1), lambda g: (0, 0)),
        ],
        out_specs=pl.BlockSpec((GB, 1), lambda g: (g, 0)),
        out_shape=jax.ShapeDtypeStruct((G, 1), jnp.float32),
    )(parts, self80, b1.reshape(1, HID), w2t, lin_b.reshape(1, 1))
    return out
